# baseline (speedup 1.0000x reference)
import functools

import jax
import jax.numpy as jnp
from jax.experimental import pallas as pl
from jax.experimental.pallas import tpu as pltpu


def _router_kernel(x_ref, w_ref, b_ref, out_ref, cnt_ref):
    acc = jnp.dot(x_ref[...], w_ref[...], preferred_element_type=jnp.float32)
    logits = jnp.maximum(acc + b_ref[...], 0.0)
    out_ref[...] = logits
    nz = jnp.sum((logits > 0.0).astype(jnp.float32))
    cnt_ref[...] = jnp.full(cnt_ref.shape, nz, dtype=jnp.float32)


@functools.partial(jax.jit, static_argnames=("block_t",))
def _run(x, W, b, block_t):
    n_tokens, d_model = x.shape
    n_experts = W.shape[1]
    n_tiles = n_tokens // block_t
    b2 = b.reshape(1, n_experts)

    logits, counts = pl.pallas_call(
        _router_kernel,
        grid=(n_tiles,),
        in_specs=[
            pl.BlockSpec((block_t, d_model), lambda i: (i, 0)),
            pl.BlockSpec((d_model, n_experts), lambda i: (0, 0)),
            pl.BlockSpec((1, n_experts), lambda i: (0, 0)),
        ],
        out_specs=[
            pl.BlockSpec((block_t, n_experts), lambda i: (i, 0)),
            pl.BlockSpec((1, 1, 128), lambda i: (i, 0, 0)),
        ],
        out_shape=[
            jax.ShapeDtypeStruct((n_tokens, n_experts), jnp.float32),
            jax.ShapeDtypeStruct((n_tiles, 1, 128), jnp.float32),
        ],
        compiler_params=pltpu.CompilerParams(
            dimension_semantics=("parallel",)
        ),
    )(x, W, b2)

    density = jnp.sum(counts[:, 0, 0]) / (n_tokens * n_experts)
    return logits, density.astype(jnp.float32)


def kernel(x, W, b):
    return _run(x, W, b, 512)


# speedup vs baseline: 1.0030x; 1.0030x over previous
import functools

import jax
import jax.numpy as jnp
from jax.experimental import pallas as pl
from jax.experimental.pallas import tpu as pltpu


def _router_kernel(x_ref, w_ref, b_ref, out_ref, cnt_ref):
    acc = jnp.dot(
        x_ref[...].astype(jnp.bfloat16),
        w_ref[...],
        preferred_element_type=jnp.float32,
    )
    logits = jnp.maximum(acc + b_ref[...], 0.0)
    out_ref[...] = logits
    nz = jnp.sum((logits > 0.0).astype(jnp.float32))
    cnt_ref[...] = jnp.full(cnt_ref.shape, nz, dtype=jnp.float32)


@functools.partial(jax.jit, static_argnames=("block_t",))
def _run(x, W, b, block_t):
    n_tokens, d_model = x.shape
    n_experts = W.shape[1]
    n_tiles = n_tokens // block_t
    b2 = b.reshape(1, n_experts)
    Wb = W.astype(jnp.bfloat16)

    logits, counts = pl.pallas_call(
        _router_kernel,
        grid=(n_tiles,),
        in_specs=[
            pl.BlockSpec((block_t, d_model), lambda i: (i, 0)),
            pl.BlockSpec((d_model, n_experts), lambda i: (0, 0)),
            pl.BlockSpec((1, n_experts), lambda i: (0, 0)),
        ],
        out_specs=[
            pl.BlockSpec((block_t, n_experts), lambda i: (i, 0)),
            pl.BlockSpec((1, 1, 128), lambda i: (i, 0, 0)),
        ],
        out_shape=[
            jax.ShapeDtypeStruct((n_tokens, n_experts), jnp.float32),
            jax.ShapeDtypeStruct((n_tiles, 1, 128), jnp.float32),
        ],
        compiler_params=pltpu.CompilerParams(
            dimension_semantics=("parallel",)
        ),
    )(x, Wb, b2)

    density = jnp.sum(counts[:, 0, 0]) / (n_tokens * n_experts)
    return logits, density.astype(jnp.float32)


def kernel(x, W, b):
    return _run(x, W, b, 512)


# speedup vs baseline: 1.0089x; 1.0059x over previous
import functools

import jax
import jax.numpy as jnp
from jax.experimental import pallas as pl
from jax.experimental.pallas import tpu as pltpu


def _router_kernel(x_ref, w_ref, b_ref, out_ref, cnt_ref):
    acc = jnp.dot(
        x_ref[...].astype(jnp.bfloat16),
        w_ref[...],
        preferred_element_type=jnp.float32,
    )
    logits = jnp.maximum(acc + b_ref[...], 0.0)
    out_ref[...] = logits
    nz = jnp.sum((logits > 0.0).astype(jnp.float32))
    cnt_ref[...] = jnp.full(cnt_ref.shape, nz, dtype=jnp.float32)


@functools.partial(jax.jit, static_argnames=("block_t",))
def _run(x, W, b, block_t):
    n_tokens, d_model = x.shape
    n_experts = W.shape[1]
    n_tiles = n_tokens // block_t
    b2 = b.reshape(1, n_experts)
    Wb = W.astype(jnp.bfloat16)

    logits, counts = pl.pallas_call(
        _router_kernel,
        grid=(n_tiles,),
        in_specs=[
            pl.BlockSpec((block_t, d_model), lambda i: (i, 0)),
            pl.BlockSpec((d_model, n_experts), lambda i: (0, 0)),
            pl.BlockSpec((1, n_experts), lambda i: (0, 0)),
        ],
        out_specs=[
            pl.BlockSpec((block_t, n_experts), lambda i: (i, 0)),
            pl.BlockSpec((1, 1, 128), lambda i: (i, 0, 0)),
        ],
        out_shape=[
            jax.ShapeDtypeStruct((n_tokens, n_experts), jnp.float32),
            jax.ShapeDtypeStruct((n_tiles, 1, 128), jnp.float32),
        ],
        compiler_params=pltpu.CompilerParams(
            dimension_semantics=("parallel",)
        ),
    )(x, Wb, b2)

    density = jnp.sum(counts[:, 0, 0]) / (n_tokens * n_experts)
    return logits, density.astype(jnp.float32)


def kernel(x, W, b):
    return _run(x, W, b, 1024)
